# baseline (speedup 1.0000x reference)
import functools

import jax
import jax.numpy as jnp
from jax import lax
from jax.experimental import pallas as pl
from jax.experimental.pallas import tpu as pltpu
from jax.experimental.pallas import tpu_sc as plsc

N = 10000
E = 320000
DIN = 128
DH = 512
DOUT = 256

NC = 2
NS = 16
LB = 128
EP = ((E + 6 * NS * LB - 1) // (6 * NS * LB)) * (6 * NS * LB)
NB1 = EP // (NC * NS * LB)
NB2 = EP // (NS * LB)
NP = 10240
RPT = NP // NS

_HIGH = jax.lax.Precision.HIGHEST


def _mesh():
    return plsc.VectorSubcoreMesh(core_axis_name="c", subcore_axis_name="s")


@functools.partial(
    pl.kernel,
    out_type=jax.ShapeDtypeStruct((NC * NS * N,), jnp.float32),
    mesh=_mesh(),
    compiler_params=pltpu.CompilerParams(needs_layout_passes=False, use_tc_tiling_on_sc=False),
    scratch_types=[
        pltpu.VMEM((NB1, LB), jnp.int32),
        pltpu.VMEM((NB1, LB), jnp.float32),
        pltpu.VMEM((N,), jnp.float32),
    ],
)
def _k_deg(d_idx, e_w, out, dst_v, ew_v, deg_v):
    c = lax.axis_index("c")
    s = lax.axis_index("s")
    pltpu.sync_copy(d_idx.at[c, s], dst_v)
    pltpu.sync_copy(e_w.at[c, s], ew_v)

    def zb(i, carry):
        deg_v[pl.ds(i * 16, 16)] = jnp.zeros((16,), jnp.float32)
        return carry

    lax.fori_loop(0, N // 16, zb, 0)

    def bb(b, carry):
        def gb(g, c3):
            dd = dst_v[b, pl.ds(g * 16, 16)]
            ww = ew_v[b, pl.ds(g * 16, 16)]
            plsc.addupdate_scatter(deg_v, [dd], ww)
            return c3

        return lax.fori_loop(0, LB // 16, gb, carry)

    lax.fori_loop(0, NB1, bb, 0)
    wid = c * NS + s
    pltpu.sync_copy(deg_v, out.at[pl.ds(wid * N, N)])


DC = 64


def _make_pass(nk):
    @functools.partial(
        pl.kernel,
        out_type=jax.ShapeDtypeStruct((nk, NC, NP, DC), jnp.float32),
        mesh=_mesh(),
        compiler_params=pltpu.CompilerParams(
            needs_layout_passes=False, use_tc_tiling_on_sc=False),
        scratch_types=[
            pltpu.VMEM((NB2, LB), jnp.int32),
            pltpu.VMEM((NB2, LB), jnp.int32),
            pltpu.VMEM((NB2, LB), jnp.float32),
            pltpu.VMEM((3, LB, DC), jnp.float32),
            pltpu.VMEM_SHARED((NP, DC), jnp.float32),
            pltpu.SemaphoreType.DMA((3,)),
        ],
    )
    def kern(table, s_idx, d_idx, e_w, zeros, out,
             src_v, dst_v, ew_v, rows_v, acc, gsem_a):
        gsem = [gsem_a.at[r] for r in range(3)]
        c = lax.axis_index("c")
        s = lax.axis_index("s")
        pltpu.sync_copy(d_idx.at[s], dst_v)
        pltpu.sync_copy(e_w.at[s], ew_v)

        def g_start(b, r):
            pltpu.async_copy(table.at[src_v.at[b]], rows_v.at[r], gsem[r])

        def g_wait(b, r):
            pltpu.make_async_copy(
                table.at[src_v.at[b]], rows_v.at[r], gsem[r]).wait()

        for k in range(nk):
            pltpu.sync_copy(s_idx.at[k, c, s], src_v)
            pltpu.sync_copy(zeros.at[pl.ds(s * RPT, RPT)],
                            acc.at[pl.ds(s * RPT, RPT)])
            plsc.subcore_barrier()

            g_start(0, 0)
            g_start(1, 1)

            def trip(p, carry):
                for u in range(3):
                    b = 3 * p + u
                    g_wait(b, u)

                    @plsc.parallel_loop(0, LB // 16, unroll=2)
                    def escale(g16):
                        wv = ew_v[b, pl.ds(g16 * 16, 16)]
                        for j in range(16):
                            w = wv[j]
                            e_row = g16 * 16 + j
                            for g in range(DC // 16):
                                sl = pl.ds(g * 16, 16)
                                rows_v[u, e_row, sl] = rows_v[u, e_row, sl] * w
                    pltpu.sync_copy(rows_v.at[u], acc.at[dst_v.at[b]], add=True)

                    @pl.when(b + 2 < NB2)
                    def _():
                        g_start(b + 2, (u + 2) % 3)
                return carry

            lax.fori_loop(0, NB2 // 3, trip, 0)
            plsc.subcore_barrier()
            pltpu.sync_copy(acc.at[pl.ds(s * RPT, RPT)],
                            out.at[k, c, pl.ds(s * RPT, RPT)])

    return kern


_k_pass1 = _make_pass(1)
_k_pass2 = _make_pass(2)


RB = 2000
GR = N // RB


def _row_spec(cols):
    return pl.BlockSpec((RB, cols), lambda i: (i, 0))


def _full_spec(rows, cols):
    return pl.BlockSpec((rows, cols), lambda i: (0, 0))


def _k_dinv(degp):
    def body(degp_ref, out_ref):
        deg = jnp.sum(degp_ref[...], axis=0) + 1.0
        out_ref[...] = jax.lax.rsqrt(jnp.maximum(deg, 1e-12))[None, :]

    return pl.pallas_call(
        body, out_shape=jax.ShapeDtypeStruct((1, N), jnp.float32))(degp)


def _k_xs(x, dinv_col):
    def body(x_ref, dv_ref, out_ref):
        out_ref[...] = x_ref[...] * dv_ref[...]

    return pl.pallas_call(
        body,
        grid=(GR,),
        in_specs=[_row_spec(DIN), _row_spec(1)],
        out_specs=_row_spec(DIN),
        out_shape=jax.ShapeDtypeStruct((N, DIN), jnp.float32),
    )(x, dinv_col)


def _k_mid_a(sc1, xs, dinv_col, W1, b1):
    def body(sc1_ref, xs_ref, dv_ref, w_ref, b_ref, out_ref, st_ref):
        agg = (sc1_ref[...] + xs_ref[...]) * dv_ref[...]
        blk = jnp.dot(agg, w_ref[...], precision=_HIGH) + b_ref[...]
        out_ref[...] = blk

        @pl.when(pl.program_id(0) == 0)
        def _():
            st_ref[...] = jnp.zeros_like(st_ref)

        na = (pl.program_id(0) * RB).astype(jnp.float32)
        nt = na + RB
        mb = jnp.mean(blk, axis=0, keepdims=True)
        m2b = jnp.sum((blk - mb) * (blk - mb), axis=0, keepdims=True)
        ma = st_ref[0:1, :]
        delta = mb - ma
        st_ref[0:1, :] = ma + delta * (RB / nt)
        st_ref[1:2, :] += m2b + delta * delta * (na * RB / nt)

    return pl.pallas_call(
        body,
        grid=(GR,),
        in_specs=[_row_spec(DIN), _row_spec(DIN), _row_spec(1),
                  _full_spec(DIN, DH), _full_spec(1, DH)],
        out_specs=(_row_spec(DH), _full_spec(8, DH)),
        out_shape=(jax.ShapeDtypeStruct((N, DH), jnp.float32),
                   jax.ShapeDtypeStruct((8, DH), jnp.float32)),
    )(sc1, xs, dinv_col, W1, b1[None, :])


def _k_mid_c(pre1, st1, g1, bb1, W2, dinv_col):
    def body(h_ref, st_ref, g_ref, b_ref, w_ref, dv_ref, out_ref):
        m = st_ref[0:1, :]
        v = st_ref[1:2, :] * (1.0 / N)
        h1 = jnp.maximum(
            (h_ref[...] - m) * jax.lax.rsqrt(v + 1e-5) * g_ref[...]
            + b_ref[...], 0.0)
        out_ref[...] = (
            jnp.dot(h1, w_ref[...], precision=_HIGH) * dv_ref[...]
        )

    return pl.pallas_call(
        body,
        grid=(GR,),
        in_specs=[_row_spec(DH), _full_spec(8, DH), _full_spec(1, DH),
                  _full_spec(1, DH), _full_spec(DH, DOUT), _row_spec(1)],
        out_specs=_row_spec(DOUT),
        out_shape=jax.ShapeDtypeStruct((N, DOUT), jnp.float32),
    )(pre1, st1, g1[None, :], bb1[None, :], W2, dinv_col)


def _k_feat_stats(f):
    def body(f_ref, s1_ref, s2_ref):
        fv = f_ref[...]
        s1_ref[...] = jnp.sum(fv)[None, None]
        s2_ref[...] = jnp.sum(fv * fv)[None, None]

    return pl.pallas_call(
        body,
        out_shape=(jax.ShapeDtypeStruct((1, 1), jnp.float32),
                   jax.ShapeDtypeStruct((1, 1), jnp.float32)),
    )(f)


def _k_head(dist_feat, degree_feat, ds1, ds2, gs1, gs2,
            Wd, bd, bnd_g, bnd_b, Wg, bg, bng_g, bng_b, Wm1, Wm2):
    def body(df_ref, gf_ref, ds1_ref, ds2_ref, gs1_ref, gs2_ref,
             wd_ref, bd_ref, dgam_ref, dbet_ref,
             wg_ref, bg_ref, ggam_ref, gbet_ref, wm1_ref, wm2_ref, out_ref):
        def head(f, s1, s2, w, b, gamma, beta):
            mu = s1[0, 0] * (1.0 / N)
            e2 = s2[0, 0] * (1.0 / N)
            m = mu * w + b
            v = (e2 - mu * mu) * (w * w)
            h = f * w + b
            return jnp.maximum(
                (h - m) * jax.lax.rsqrt(v + 1e-5) * gamma + beta, 0.0)

        d = head(df_ref[...], ds1_ref, ds2_ref, wd_ref[...], bd_ref[...],
                 dgam_ref[...], dbet_ref[...])
        g = head(gf_ref[...], gs1_ref, gs2_ref, wg_ref[...], bg_ref[...],
                 ggam_ref[...], gbet_ref[...])
        out_ref[...] = jnp.dot(d, wm1_ref[...], precision=_HIGH) + jnp.dot(
            g, wm2_ref[...], precision=_HIGH)

    return pl.pallas_call(
        body,
        grid=(GR,),
        in_specs=[_row_spec(1), _row_spec(1),
                  _full_spec(1, 1), _full_spec(1, 1),
                  _full_spec(1, 1), _full_spec(1, 1),
                  _full_spec(1, DOUT), _full_spec(1, DOUT),
                  _full_spec(1, DOUT), _full_spec(1, DOUT),
                  _full_spec(1, DOUT), _full_spec(1, DOUT),
                  _full_spec(1, DOUT), _full_spec(1, DOUT),
                  _full_spec(DOUT, DOUT), _full_spec(DOUT, DOUT)],
        out_specs=_row_spec(DOUT),
        out_shape=jax.ShapeDtypeStruct((N, DOUT), jnp.float32),
    )(dist_feat, degree_feat, ds1, ds2, gs1, gs2,
      Wd, bd[None, :], bnd_g[None, :], bnd_b[None, :],
      Wg, bg[None, :], bng_g[None, :], bng_b[None, :], Wm1, Wm2)


def _k_h2pre(sc2, t2, dinv_col, b2):
    def body(sc2_ref, t2_ref, dv_ref, b_ref, out_ref, st_ref):
        blk = (sc2_ref[...] + t2_ref[...]) * dv_ref[...] + b_ref[...]
        out_ref[...] = blk

        @pl.when(pl.program_id(0) == 0)
        def _():
            st_ref[...] = jnp.zeros_like(st_ref)

        na = (pl.program_id(0) * RB).astype(jnp.float32)
        nt = na + RB
        mb = jnp.mean(blk, axis=0, keepdims=True)
        m2b = jnp.sum((blk - mb) * (blk - mb), axis=0, keepdims=True)
        ma = st_ref[0:1, :]
        delta = mb - ma
        st_ref[0:1, :] = ma + delta * (RB / nt)
        st_ref[1:2, :] += m2b + delta * delta * (na * RB / nt)

    return pl.pallas_call(
        body,
        grid=(GR,),
        in_specs=[_row_spec(DOUT), _row_spec(DOUT), _row_spec(1),
                  _full_spec(1, DOUT)],
        out_specs=(_row_spec(DOUT), _full_spec(8, DOUT)),
        out_shape=(jax.ShapeDtypeStruct((N, DOUT), jnp.float32),
                   jax.ShapeDtypeStruct((8, DOUT), jnp.float32)),
    )(sc2, t2, dinv_col, b2[None, :])


def _k_embs(h2pre, st2, g2, bb2, Wm0, bm, dg):
    def body(h_ref, st_ref, g_ref, b_ref, w_ref, bm_ref, dg_ref, out_ref):
        m = st_ref[0:1, :]
        v = st_ref[1:2, :] * (1.0 / N)
        h2 = jnp.maximum(
            (h_ref[...] - m) * jax.lax.rsqrt(v + 1e-5) * g_ref[...]
            + b_ref[...], 0.0)
        e = (jnp.dot(h2, w_ref[...], precision=_HIGH)
             + dg_ref[...] + bm_ref[...])
        nrm = jnp.sqrt(jnp.sum(e * e, axis=1, keepdims=True))
        out_ref[...] = e / jnp.maximum(nrm, 1e-12)

    return pl.pallas_call(
        body,
        grid=(GR,),
        in_specs=[_row_spec(DOUT), _full_spec(8, DOUT), _full_spec(1, DOUT),
                  _full_spec(1, DOUT), _full_spec(DOUT, DOUT),
                  _full_spec(1, DOUT), _row_spec(DOUT)],
        out_specs=_row_spec(DOUT),
        out_shape=jax.ShapeDtypeStruct((N, DOUT), jnp.float32),
    )(h2pre, st2, g2[None, :], bb2[None, :], Wm0, bm[None, :], dg)


def _k_heads_out(embs, fW1, fb1, fW2r, fb2, cW1, cb1, cW2r, cb2):
    def body(e_ref, fw1_ref, fb1_ref, fw2_ref, fb2_ref,
             cw1_ref, cb1_ref, cw2_ref, cb2_ref, fac_ref, cov_ref):
        e = e_ref[...]
        f = jnp.maximum(
            jnp.dot(e, fw1_ref[...], precision=_HIGH) + fb1_ref[...], 0.0)
        fac_ref[...] = (
            jnp.sum(f * fw2_ref[...], axis=1, keepdims=True) + fb2_ref[...]
        )
        cc = jnp.maximum(
            jnp.dot(e, cw1_ref[...], precision=_HIGH) + cb1_ref[...], 0.0)
        logit = jnp.sum(cc * cw2_ref[...], axis=1, keepdims=True) + cb2_ref[...]
        cov_ref[...] = 1.0 / (1.0 + jnp.exp(-logit))

    return pl.pallas_call(
        body,
        grid=(GR,),
        in_specs=[_row_spec(DOUT),
                  _full_spec(DOUT, DH // 2), _full_spec(1, DH // 2),
                  _full_spec(1, DH // 2), _full_spec(1, 1),
                  _full_spec(DOUT, DH // 2), _full_spec(1, DH // 2),
                  _full_spec(1, DH // 2), _full_spec(1, 1)],
        out_specs=(_row_spec(1), _row_spec(1)),
        out_shape=(jax.ShapeDtypeStruct((N, 1), jnp.float32),
                   jax.ShapeDtypeStruct((N, 1), jnp.float32)),
    )(embs, fW1, fb1[None, :], fW2r, fb2[None, :],
      cW1, cb1[None, :], cW2r, cb2[None, :])


def kernel(idx, x, edge_index, edge_weight, dist_feat, degree_feat, batch_size,
           W1, b1, bn1_g, bn1_b, W2, b2, bn2_g, bn2_b,
           Wd, bd, bnd_g, bnd_b, Wg, bg, bng_g, bng_b,
           Wm, bm, fW1, fb1, fW2, fb2, cW1, cb1, cW2, cb2):
    src = edge_index[0]
    dst = edge_index[1]
    e = src.shape[0]
    pad = EP - e
    srcp = jnp.concatenate([src, jnp.zeros((pad,), src.dtype)])
    dstp = jnp.concatenate([dst, jnp.zeros((pad,), dst.dtype)])
    ewp = jnp.concatenate([edge_weight, jnp.zeros((pad,), edge_weight.dtype)])

    d1 = dstp.reshape(NC, NS, NB1, LB)
    w1e = ewp.reshape(NC, NS, NB1, LB)

    s16 = srcp.reshape(NS, NB2, LB)
    d16 = dstp.reshape(NS, NB2, LB)
    w16 = ewp.reshape(NS, NB2, LB)

    zeros = jnp.zeros((NP, DC), jnp.float32)

    degp = _k_deg(d1, w1e).reshape(NC * NS, N)
    dinv_col = _k_dinv(degp).reshape(N, 1)
    xs = _k_xs(x, dinv_col)
    t1 = jnp.concatenate([xs[:, :DC], xs[:, DC:]], axis=0)
    s1k = jnp.stack([s16, s16 + N])[None]
    o1 = _k_pass1(t1, s1k, d16, w16, zeros)
    sc1 = jnp.concatenate([o1[0, 0, :N], o1[0, 1, :N]], axis=1)
    pre1, st1 = _k_mid_a(sc1, xs, dinv_col, W1, b1)
    t2 = _k_mid_c(pre1, st1, bn1_g, bn1_b, W2, dinv_col)
    t2s = jnp.concatenate([t2[:, m * DC:(m + 1) * DC] for m in range(4)], axis=0)
    s2k = jnp.stack([jnp.stack([s16 + (k * 2 + cc) * N for cc in range(2)])
                     for k in range(2)])
    o2 = _k_pass2(t2s, s2k, d16, w16, zeros)
    sc2 = jnp.concatenate(
        [o2[0, 0, :N], o2[0, 1, :N], o2[1, 0, :N], o2[1, 1, :N]], axis=1)
    ds1, ds2 = _k_feat_stats(dist_feat)
    gs1, gs2 = _k_feat_stats(degree_feat)
    dg = _k_head(dist_feat, degree_feat, ds1, ds2, gs1, gs2,
                 Wd, bd, bnd_g, bnd_b,
                 Wg, bg, bng_g, bng_b, Wm[DOUT:2 * DOUT], Wm[2 * DOUT:])
    h2pre, st2 = _k_h2pre(sc2, t2, dinv_col, b2)
    embs = _k_embs(h2pre, st2, bn2_g, bn2_b, Wm[:DOUT], bm, dg)
    fac, cov = _k_heads_out(embs, fW1, fb1, fW2.reshape(1, -1), fb2,
                            cW1, cb1, cW2.reshape(1, -1), cb2)
    return (embs, fac, cov)


# speedup vs baseline: 1.0184x; 1.0184x over previous
import functools

import jax
import jax.numpy as jnp
from jax import lax
from jax.experimental import pallas as pl
from jax.experimental.pallas import tpu as pltpu
from jax.experimental.pallas import tpu_sc as plsc

N = 10000
E = 320000
DIN = 128
DH = 512
DOUT = 256

NC = 2
NS = 16
LB = 128
EP = ((E + 6 * NS * LB - 1) // (6 * NS * LB)) * (6 * NS * LB)
NB1 = EP // (NC * NS * LB)
NB2 = EP // (NS * LB)
NP = 10240
RPT = NP // NS

_HIGH = jax.lax.Precision.HIGHEST


def _mesh():
    return plsc.VectorSubcoreMesh(core_axis_name="c", subcore_axis_name="s")


@functools.partial(
    pl.kernel,
    out_type=jax.ShapeDtypeStruct((NC * NS * N,), jnp.float32),
    mesh=_mesh(),
    compiler_params=pltpu.CompilerParams(needs_layout_passes=False, use_tc_tiling_on_sc=False),
    scratch_types=[
        pltpu.VMEM((NB1, LB), jnp.int32),
        pltpu.VMEM((NB1, LB), jnp.float32),
        pltpu.VMEM((N,), jnp.float32),
    ],
)
def _k_deg(d_idx, e_w, out, dst_v, ew_v, deg_v):
    c = lax.axis_index("c")
    s = lax.axis_index("s")
    pltpu.sync_copy(d_idx.at[c, s], dst_v)
    pltpu.sync_copy(e_w.at[c, s], ew_v)

    def zb(i, carry):
        deg_v[pl.ds(i * 16, 16)] = jnp.zeros((16,), jnp.float32)
        return carry

    lax.fori_loop(0, N // 16, zb, 0)

    def bb(b, carry):
        def gb(g, c3):
            dd = dst_v[b, pl.ds(g * 16, 16)]
            ww = ew_v[b, pl.ds(g * 16, 16)]
            plsc.addupdate_scatter(deg_v, [dd], ww)
            return c3

        return lax.fori_loop(0, LB // 16, gb, carry)

    lax.fori_loop(0, NB1, bb, 0)
    wid = c * NS + s
    pltpu.sync_copy(deg_v, out.at[pl.ds(wid * N, N)])


DC = 64


def _make_pass(nk):
    @functools.partial(
        pl.kernel,
        out_type=jax.ShapeDtypeStruct((nk, NC, NP, DC), jnp.float32),
        mesh=_mesh(),
        compiler_params=pltpu.CompilerParams(
            needs_layout_passes=False, use_tc_tiling_on_sc=False),
        scratch_types=[
            pltpu.VMEM((NB2, LB), jnp.int32),
            pltpu.VMEM((NB2, LB), jnp.int32),
            pltpu.VMEM((NB2, LB), jnp.float32),
            pltpu.VMEM((3, LB, DC), jnp.float32),
            pltpu.VMEM_SHARED((NP, DC), jnp.float32),
            pltpu.SemaphoreType.DMA((3,)),
            pltpu.SemaphoreType.DMA((3,)),
        ],
    )
    def kern(table, s_idx, d_idx, e_w, zeros, out,
             src_v, dst_v, ew_v, rows_v, acc, gsem_a, ssem_a):
        gsem = [gsem_a.at[r] for r in range(3)]
        ssem = [ssem_a.at[r] for r in range(3)]
        c = lax.axis_index("c")
        s = lax.axis_index("s")
        pltpu.sync_copy(d_idx.at[s], dst_v)
        pltpu.sync_copy(e_w.at[s], ew_v)

        def g_start(b, r):
            pltpu.async_copy(table.at[src_v.at[b]], rows_v.at[r], gsem[r])

        def g_wait(b, r):
            pltpu.make_async_copy(
                table.at[src_v.at[b]], rows_v.at[r], gsem[r]).wait()

        def s_start(b, r):
            pltpu.async_copy(
                rows_v.at[r], acc.at[dst_v.at[b]], ssem[r], add=True)

        def s_wait(b, r):
            pltpu.make_async_copy(
                rows_v.at[r], acc.at[dst_v.at[b]], ssem[r]).wait()

        for k in range(nk):
            pltpu.sync_copy(s_idx.at[k, c, s], src_v)
            pltpu.sync_copy(zeros.at[pl.ds(s * RPT, RPT)],
                            acc.at[pl.ds(s * RPT, RPT)])
            plsc.subcore_barrier()

            g_start(0, 0)
            g_start(1, 1)

            def trip(p, carry):
                for u in range(3):
                    b = 3 * p + u
                    g_wait(b, u)

                    @plsc.parallel_loop(0, LB // 16, unroll=2)
                    def escale(g16):
                        wv = ew_v[b, pl.ds(g16 * 16, 16)]
                        for j in range(16):
                            w = wv[j]
                            e_row = g16 * 16 + j
                            for g in range(DC // 16):
                                sl = pl.ds(g * 16, 16)
                                rows_v[u, e_row, sl] = rows_v[u, e_row, sl] * w
                    s_start(b, u)

                    @pl.when(b + 2 < NB2)
                    def _():
                        @pl.when(b >= 1)
                        def _():
                            s_wait(b - 1, (u + 2) % 3)

                        g_start(b + 2, (u + 2) % 3)
                return carry

            lax.fori_loop(0, NB2 // 3, trip, 0)
            s_wait(NB2 - 3, (NB2 - 3) % 3)
            s_wait(NB2 - 2, (NB2 - 2) % 3)
            s_wait(NB2 - 1, (NB2 - 1) % 3)
            plsc.subcore_barrier()
            pltpu.sync_copy(acc.at[pl.ds(s * RPT, RPT)],
                            out.at[k, c, pl.ds(s * RPT, RPT)])

    return kern


_k_pass1 = _make_pass(1)
_k_pass2 = _make_pass(2)


RB = 2000
GR = N // RB


def _row_spec(cols):
    return pl.BlockSpec((RB, cols), lambda i: (i, 0))


def _full_spec(rows, cols):
    return pl.BlockSpec((rows, cols), lambda i: (0, 0))


def _k_dinv(degp):
    def body(degp_ref, out_ref):
        deg = jnp.sum(degp_ref[...], axis=0) + 1.0
        out_ref[...] = jax.lax.rsqrt(jnp.maximum(deg, 1e-12))[None, :]

    return pl.pallas_call(
        body, out_shape=jax.ShapeDtypeStruct((1, N), jnp.float32))(degp)


def _k_xs(x, dinv_col):
    def body(x_ref, dv_ref, out_ref):
        out_ref[...] = x_ref[...] * dv_ref[...]

    return pl.pallas_call(
        body,
        grid=(GR,),
        in_specs=[_row_spec(DIN), _row_spec(1)],
        out_specs=_row_spec(DIN),
        out_shape=jax.ShapeDtypeStruct((N, DIN), jnp.float32),
    )(x, dinv_col)


def _k_mid_a(sc1, xs, dinv_col, W1, b1):
    def body(sc1_ref, xs_ref, dv_ref, w_ref, b_ref, out_ref, st_ref):
        agg = (sc1_ref[...] + xs_ref[...]) * dv_ref[...]
        blk = jnp.dot(agg, w_ref[...], precision=_HIGH) + b_ref[...]
        out_ref[...] = blk

        @pl.when(pl.program_id(0) == 0)
        def _():
            st_ref[...] = jnp.zeros_like(st_ref)

        na = (pl.program_id(0) * RB).astype(jnp.float32)
        nt = na + RB
        mb = jnp.mean(blk, axis=0, keepdims=True)
        m2b = jnp.sum((blk - mb) * (blk - mb), axis=0, keepdims=True)
        ma = st_ref[0:1, :]
        delta = mb - ma
        st_ref[0:1, :] = ma + delta * (RB / nt)
        st_ref[1:2, :] += m2b + delta * delta * (na * RB / nt)

    return pl.pallas_call(
        body,
        grid=(GR,),
        in_specs=[_row_spec(DIN), _row_spec(DIN), _row_spec(1),
                  _full_spec(DIN, DH), _full_spec(1, DH)],
        out_specs=(_row_spec(DH), _full_spec(8, DH)),
        out_shape=(jax.ShapeDtypeStruct((N, DH), jnp.float32),
                   jax.ShapeDtypeStruct((8, DH), jnp.float32)),
    )(sc1, xs, dinv_col, W1, b1[None, :])


def _k_mid_c(pre1, st1, g1, bb1, W2, dinv_col):
    def body(h_ref, st_ref, g_ref, b_ref, w_ref, dv_ref, out_ref):
        m = st_ref[0:1, :]
        v = st_ref[1:2, :] * (1.0 / N)
        h1 = jnp.maximum(
            (h_ref[...] - m) * jax.lax.rsqrt(v + 1e-5) * g_ref[...]
            + b_ref[...], 0.0)
        out_ref[...] = (
            jnp.dot(h1, w_ref[...], precision=_HIGH) * dv_ref[...]
        )

    return pl.pallas_call(
        body,
        grid=(GR,),
        in_specs=[_row_spec(DH), _full_spec(8, DH), _full_spec(1, DH),
                  _full_spec(1, DH), _full_spec(DH, DOUT), _row_spec(1)],
        out_specs=_row_spec(DOUT),
        out_shape=jax.ShapeDtypeStruct((N, DOUT), jnp.float32),
    )(pre1, st1, g1[None, :], bb1[None, :], W2, dinv_col)


def _k_feat_stats(f):
    def body(f_ref, s1_ref, s2_ref):
        fv = f_ref[...]
        s1_ref[...] = jnp.sum(fv)[None, None]
        s2_ref[...] = jnp.sum(fv * fv)[None, None]

    return pl.pallas_call(
        body,
        out_shape=(jax.ShapeDtypeStruct((1, 1), jnp.float32),
                   jax.ShapeDtypeStruct((1, 1), jnp.float32)),
    )(f)


def _k_head(dist_feat, degree_feat, ds1, ds2, gs1, gs2,
            Wd, bd, bnd_g, bnd_b, Wg, bg, bng_g, bng_b, Wm1, Wm2):
    def body(df_ref, gf_ref, ds1_ref, ds2_ref, gs1_ref, gs2_ref,
             wd_ref, bd_ref, dgam_ref, dbet_ref,
             wg_ref, bg_ref, ggam_ref, gbet_ref, wm1_ref, wm2_ref, out_ref):
        def head(f, s1, s2, w, b, gamma, beta):
            mu = s1[0, 0] * (1.0 / N)
            e2 = s2[0, 0] * (1.0 / N)
            m = mu * w + b
            v = (e2 - mu * mu) * (w * w)
            h = f * w + b
            return jnp.maximum(
                (h - m) * jax.lax.rsqrt(v + 1e-5) * gamma + beta, 0.0)

        d = head(df_ref[...], ds1_ref, ds2_ref, wd_ref[...], bd_ref[...],
                 dgam_ref[...], dbet_ref[...])
        g = head(gf_ref[...], gs1_ref, gs2_ref, wg_ref[...], bg_ref[...],
                 ggam_ref[...], gbet_ref[...])
        out_ref[...] = jnp.dot(d, wm1_ref[...], precision=_HIGH) + jnp.dot(
            g, wm2_ref[...], precision=_HIGH)

    return pl.pallas_call(
        body,
        grid=(GR,),
        in_specs=[_row_spec(1), _row_spec(1),
                  _full_spec(1, 1), _full_spec(1, 1),
                  _full_spec(1, 1), _full_spec(1, 1),
                  _full_spec(1, DOUT), _full_spec(1, DOUT),
                  _full_spec(1, DOUT), _full_spec(1, DOUT),
                  _full_spec(1, DOUT), _full_spec(1, DOUT),
                  _full_spec(1, DOUT), _full_spec(1, DOUT),
                  _full_spec(DOUT, DOUT), _full_spec(DOUT, DOUT)],
        out_specs=_row_spec(DOUT),
        out_shape=jax.ShapeDtypeStruct((N, DOUT), jnp.float32),
    )(dist_feat, degree_feat, ds1, ds2, gs1, gs2,
      Wd, bd[None, :], bnd_g[None, :], bnd_b[None, :],
      Wg, bg[None, :], bng_g[None, :], bng_b[None, :], Wm1, Wm2)


def _k_h2pre(sc2, t2, dinv_col, b2):
    def body(sc2_ref, t2_ref, dv_ref, b_ref, out_ref, st_ref):
        blk = (sc2_ref[...] + t2_ref[...]) * dv_ref[...] + b_ref[...]
        out_ref[...] = blk

        @pl.when(pl.program_id(0) == 0)
        def _():
            st_ref[...] = jnp.zeros_like(st_ref)

        na = (pl.program_id(0) * RB).astype(jnp.float32)
        nt = na + RB
        mb = jnp.mean(blk, axis=0, keepdims=True)
        m2b = jnp.sum((blk - mb) * (blk - mb), axis=0, keepdims=True)
        ma = st_ref[0:1, :]
        delta = mb - ma
        st_ref[0:1, :] = ma + delta * (RB / nt)
        st_ref[1:2, :] += m2b + delta * delta * (na * RB / nt)

    return pl.pallas_call(
        body,
        grid=(GR,),
        in_specs=[_row_spec(DOUT), _row_spec(DOUT), _row_spec(1),
                  _full_spec(1, DOUT)],
        out_specs=(_row_spec(DOUT), _full_spec(8, DOUT)),
        out_shape=(jax.ShapeDtypeStruct((N, DOUT), jnp.float32),
                   jax.ShapeDtypeStruct((8, DOUT), jnp.float32)),
    )(sc2, t2, dinv_col, b2[None, :])


def _k_embs(h2pre, st2, g2, bb2, Wm0, bm, dg):
    def body(h_ref, st_ref, g_ref, b_ref, w_ref, bm_ref, dg_ref, out_ref):
        m = st_ref[0:1, :]
        v = st_ref[1:2, :] * (1.0 / N)
        h2 = jnp.maximum(
            (h_ref[...] - m) * jax.lax.rsqrt(v + 1e-5) * g_ref[...]
            + b_ref[...], 0.0)
        e = (jnp.dot(h2, w_ref[...], precision=_HIGH)
             + dg_ref[...] + bm_ref[...])
        nrm = jnp.sqrt(jnp.sum(e * e, axis=1, keepdims=True))
        out_ref[...] = e / jnp.maximum(nrm, 1e-12)

    return pl.pallas_call(
        body,
        grid=(GR,),
        in_specs=[_row_spec(DOUT), _full_spec(8, DOUT), _full_spec(1, DOUT),
                  _full_spec(1, DOUT), _full_spec(DOUT, DOUT),
                  _full_spec(1, DOUT), _row_spec(DOUT)],
        out_specs=_row_spec(DOUT),
        out_shape=jax.ShapeDtypeStruct((N, DOUT), jnp.float32),
    )(h2pre, st2, g2[None, :], bb2[None, :], Wm0, bm[None, :], dg)


def _k_heads_out(embs, fW1, fb1, fW2r, fb2, cW1, cb1, cW2r, cb2):
    def body(e_ref, fw1_ref, fb1_ref, fw2_ref, fb2_ref,
             cw1_ref, cb1_ref, cw2_ref, cb2_ref, fac_ref, cov_ref):
        e = e_ref[...]
        f = jnp.maximum(
            jnp.dot(e, fw1_ref[...], precision=_HIGH) + fb1_ref[...], 0.0)
        fac_ref[...] = (
            jnp.sum(f * fw2_ref[...], axis=1, keepdims=True) + fb2_ref[...]
        )
        cc = jnp.maximum(
            jnp.dot(e, cw1_ref[...], precision=_HIGH) + cb1_ref[...], 0.0)
        logit = jnp.sum(cc * cw2_ref[...], axis=1, keepdims=True) + cb2_ref[...]
        cov_ref[...] = 1.0 / (1.0 + jnp.exp(-logit))

    return pl.pallas_call(
        body,
        grid=(GR,),
        in_specs=[_row_spec(DOUT),
                  _full_spec(DOUT, DH // 2), _full_spec(1, DH // 2),
                  _full_spec(1, DH // 2), _full_spec(1, 1),
                  _full_spec(DOUT, DH // 2), _full_spec(1, DH // 2),
                  _full_spec(1, DH // 2), _full_spec(1, 1)],
        out_specs=(_row_spec(1), _row_spec(1)),
        out_shape=(jax.ShapeDtypeStruct((N, 1), jnp.float32),
                   jax.ShapeDtypeStruct((N, 1), jnp.float32)),
    )(embs, fW1, fb1[None, :], fW2r, fb2[None, :],
      cW1, cb1[None, :], cW2r, cb2[None, :])


def kernel(idx, x, edge_index, edge_weight, dist_feat, degree_feat, batch_size,
           W1, b1, bn1_g, bn1_b, W2, b2, bn2_g, bn2_b,
           Wd, bd, bnd_g, bnd_b, Wg, bg, bng_g, bng_b,
           Wm, bm, fW1, fb1, fW2, fb2, cW1, cb1, cW2, cb2):
    src = edge_index[0]
    dst = edge_index[1]
    e = src.shape[0]
    pad = EP - e
    srcp = jnp.concatenate([src, jnp.zeros((pad,), src.dtype)])
    dstp = jnp.concatenate([dst, jnp.zeros((pad,), dst.dtype)])
    ewp = jnp.concatenate([edge_weight, jnp.zeros((pad,), edge_weight.dtype)])

    d1 = dstp.reshape(NC, NS, NB1, LB)
    w1e = ewp.reshape(NC, NS, NB1, LB)

    s16 = srcp.reshape(NS, NB2, LB)
    d16 = dstp.reshape(NS, NB2, LB)
    w16 = ewp.reshape(NS, NB2, LB)

    zeros = jnp.zeros((NP, DC), jnp.float32)

    degp = _k_deg(d1, w1e).reshape(NC * NS, N)
    dinv_col = _k_dinv(degp).reshape(N, 1)
    xs = _k_xs(x, dinv_col)
    t1 = jnp.concatenate([xs[:, :DC], xs[:, DC:]], axis=0)
    s1k = jnp.stack([s16, s16 + N])[None]
    o1 = _k_pass1(t1, s1k, d16, w16, zeros)
    sc1 = jnp.concatenate([o1[0, 0, :N], o1[0, 1, :N]], axis=1)
    pre1, st1 = _k_mid_a(sc1, xs, dinv_col, W1, b1)
    t2 = _k_mid_c(pre1, st1, bn1_g, bn1_b, W2, dinv_col)
    t2s = jnp.concatenate([t2[:, m * DC:(m + 1) * DC] for m in range(4)], axis=0)
    s2k = jnp.stack([jnp.stack([s16 + (k * 2 + cc) * N for cc in range(2)])
                     for k in range(2)])
    o2 = _k_pass2(t2s, s2k, d16, w16, zeros)
    sc2 = jnp.concatenate(
        [o2[0, 0, :N], o2[0, 1, :N], o2[1, 0, :N], o2[1, 1, :N]], axis=1)
    ds1, ds2 = _k_feat_stats(dist_feat)
    gs1, gs2 = _k_feat_stats(degree_feat)
    dg = _k_head(dist_feat, degree_feat, ds1, ds2, gs1, gs2,
                 Wd, bd, bnd_g, bnd_b,
                 Wg, bg, bng_g, bng_b, Wm[DOUT:2 * DOUT], Wm[2 * DOUT:])
    h2pre, st2 = _k_h2pre(sc2, t2, dinv_col, b2)
    embs = _k_embs(h2pre, st2, bn2_g, bn2_b, Wm[:DOUT], bm, dg)
    fac, cov = _k_heads_out(embs, fW1, fb1, fW2.reshape(1, -1), fb2,
                            cW1, cb1, cW2.reshape(1, -1), cb2)
    return (embs, fac, cov)
